# baseline (speedup 1.0000x reference)
import functools

import jax
import jax.numpy as jnp
from jax import lax
from jax.experimental import pallas as pl
from jax.experimental.pallas import tpu as pltpu
from jax.experimental.pallas import tpu_sc as plsc

_NC = 2
_NS = 16
_NW = _NC * _NS

_CHUNK = 64


@functools.cache
def _make_gather(n_rows: int, d: int):
    assert n_rows % _NW == 0
    n_per_w = n_rows // _NW
    assert n_per_w % _CHUNK == 0
    n_chunks = n_per_w // _CHUNK
    mesh = plsc.VectorSubcoreMesh(core_axis_name="c", subcore_axis_name="s")

    _NBUF = 8
    assert n_chunks % _NBUF == 0
    n_groups = n_chunks // _NBUF

    @functools.partial(
        pl.kernel,
        out_type=jax.ShapeDtypeStruct((n_rows, d), jnp.float32),
        mesh=mesh,
        scratch_types=[
            pltpu.VMEM((n_per_w,), jnp.int32),
            pltpu.VMEM((_NBUF, _CHUNK, d), jnp.float32),
            [pltpu.SemaphoreType.DMA] * _NBUF,
            [pltpu.SemaphoreType.DMA] * _NBUF,
        ],
    )
    def gather_kernel(table_hbm, idx_hbm, out_hbm, idx_v, rows_v, g_sems, w_sems):
        wid = lax.axis_index("s") * _NC + lax.axis_index("c")
        base = wid * n_per_w
        pltpu.sync_copy(idx_hbm.at[pl.ds(base, n_per_w)], idx_v)

        def fire_g(i, buf):
            pltpu.async_copy(
                table_hbm.at[idx_v.at[pl.ds(i * _CHUNK, _CHUNK)]],
                rows_v.at[buf],
                g_sems[buf],
            )

        def drain_g(i, buf):
            pltpu.make_async_copy(
                table_hbm.at[idx_v.at[pl.ds(i * _CHUNK, _CHUNK)]],
                rows_v.at[buf],
                g_sems[buf],
            ).wait()

        def fire_w(i, buf):
            pltpu.async_copy(
                rows_v.at[buf],
                out_hbm.at[pl.ds(base + i * _CHUNK, _CHUNK)],
                w_sems[buf],
            )

        def drain_w(i, buf):
            pltpu.make_async_copy(
                rows_v.at[buf],
                out_hbm.at[pl.ds(base + i * _CHUNK, _CHUNK)],
                w_sems[buf],
            ).wait()

        for i in range(_NBUF - 1):
            fire_g(i, i)

        def step(j, carry):
            i0 = _NBUF * j
            for k in range(_NBUF):
                i = i0 + k
                bp = (k - 1) % _NBUF

                @pl.when(i >= 1)
                def _():
                    drain_w(i - 1, bp)

                @pl.when(i + _NBUF - 1 < n_chunks)
                def _():
                    fire_g(i + _NBUF - 1, bp)

                drain_g(i, k)
                fire_w(i, k)
            return carry

        lax.fori_loop(0, n_groups, step, 0)

        drain_w(n_chunks - 1, (n_chunks - 1) % _NBUF)

    return gather_kernel


def kernel(indices, table):
    b, h = indices.shape
    n_word, d = table.shape
    idx_flat = indices.reshape(-1).astype(jnp.int32)
    out = _make_gather(b * h, d)(table, idx_flat)
    return out.reshape(b, h, d)
